# initial kernel scaffold (unmeasured)
import jax
import jax.numpy as jnp
from jax import lax
from jax.experimental import pallas as pl
from jax.experimental.pallas import tpu as pltpu

N_DEV = 32
NSLOTS = 2


def kernel(x, w_mat, scale_x, scale_w):
    m_per, k = x.shape
    _, n = w_mat.shape
    n_per = n // N_DEV
    m = m_per * N_DEV

    def body(x_ref, w_ref, sx_ref, sw_ref, out_ref, acc_ref, send_sems, recv_sems):
        my = lax.axis_index("i")
        scale = sx_ref[0] * sw_ref[0]

        barrier = pltpu.get_barrier_semaphore()
        for s in range(1, N_DEV):
            peer = lax.rem(my + s, N_DEV)
            pl.semaphore_signal(
                barrier, inc=1,
                device_id=(peer,), device_id_type=pl.DeviceIdType.MESH,
            )
        pl.semaphore_wait(barrier, N_DEV - 1)

        def compute_block(dst):
            wblk = w_ref[:, pl.ds(dst * n_per, n_per)]
            acc = lax.dot_general(
                x_ref[:, :], wblk,
                dimension_numbers=(((1,), (0,)), ((), ())),
                preferred_element_type=jnp.int32,
            )
            return acc.astype(jnp.float32) * scale

        def send_desc(slot, dst):
            return pltpu.make_async_remote_copy(
                src_ref=acc_ref.at[slot],
                dst_ref=out_ref.at[pl.ds(my * m_per, m_per)],
                send_sem=send_sems.at[slot],
                recv_sem=recv_sems.at[my],
                device_id=(dst,),
                device_id_type=pl.DeviceIdType.MESH,
            )

        for s in range(1, N_DEV):
            dst = lax.rem(my + s, N_DEV)
            blk = compute_block(dst)
            slot = (s - 1) % NSLOTS
            if s - 1 >= NSLOTS:
                send_desc(slot, dst).wait_send()
            acc_ref[slot] = blk
            send_desc(slot, dst).start()

        out_ref[pl.ds(my * m_per, m_per)] = compute_block(my)

        for s in range(max(1, N_DEV - NSLOTS), N_DEV):
            dst = lax.rem(my + s, N_DEV)
            send_desc((s - 1) % NSLOTS, dst).wait_recv_and_send_dummy = None
            send_desc((s - 1) % NSLOTS, dst).wait_send()

        for s in range(1, N_DEV):
            src = lax.rem(my + s, N_DEV)
            recv = pltpu.make_async_remote_copy(
                src_ref=acc_ref.at[0],
                dst_ref=out_ref.at[pl.ds(src * m_per, m_per)],
                send_sem=send_sems.at[0],
                recv_sem=recv_sems.at[src],
                device_id=(src,),
                device_id_type=pl.DeviceIdType.MESH,
            )
            recv.wait_recv()

    return pl.pallas_call(
        body,
        out_shape=jax.ShapeDtypeStruct((m, n_per), jnp.float32),
        in_specs=[
            pl.BlockSpec(memory_space=pltpu.VMEM),
            pl.BlockSpec(memory_space=pltpu.VMEM),
            pl.BlockSpec(memory_space=pltpu.SMEM),
            pl.BlockSpec(memory_space=pltpu.SMEM),
        ],
        out_specs=pl.BlockSpec(memory_space=pltpu.VMEM),
        scratch_shapes=[
            pltpu.VMEM((NSLOTS, m_per, n_per), jnp.float32),
            pltpu.SemaphoreType.DMA((NSLOTS,)),
            pltpu.SemaphoreType.DMA((N_DEV,)),
        ],
        compiler_params=pltpu.CompilerParams(collective_id=0),
    )(x, w_mat, scale_x, scale_w)


# baseline (device time: 98689 ns/iter reference)
import jax
import jax.numpy as jnp
from jax import lax
from jax.experimental import pallas as pl
from jax.experimental.pallas import tpu as pltpu

N_DEV = 32
NSLOTS = 2


def kernel(x, w_mat, scale_x, scale_w):
    m_per, k = x.shape
    _, n = w_mat.shape
    n_per = n // N_DEV
    m = m_per * N_DEV

    def body(x_ref, w_ref, sx_ref, sw_ref, out_ref, acc_ref, send_sems, recv_sems):
        my = lax.axis_index("i")
        scale = sx_ref[0] * sw_ref[0]

        barrier = pltpu.get_barrier_semaphore()
        for s in range(1, N_DEV):
            peer = lax.rem(my + s, N_DEV)
            pl.semaphore_signal(
                barrier, inc=1,
                device_id=(peer,), device_id_type=pl.DeviceIdType.MESH,
            )
        pl.semaphore_wait(barrier, N_DEV - 1)

        def compute_block(dst):
            wblk = w_ref[:, pl.ds(dst * n_per, n_per)]
            acc = lax.dot_general(
                x_ref[:, :], wblk,
                dimension_numbers=(((1,), (0,)), ((), ())),
                preferred_element_type=jnp.int32,
            )
            return acc.astype(jnp.float32) * scale

        def send_desc(slot, dst):
            return pltpu.make_async_remote_copy(
                src_ref=acc_ref.at[slot],
                dst_ref=out_ref.at[pl.ds(my * m_per, m_per)],
                send_sem=send_sems.at[slot],
                recv_sem=recv_sems.at[my],
                device_id=(dst,),
                device_id_type=pl.DeviceIdType.MESH,
            )

        for s in range(1, N_DEV):
            dst = lax.rem(my + s, N_DEV)
            blk = compute_block(dst)
            slot = (s - 1) % NSLOTS
            if s - 1 >= NSLOTS:
                send_desc(slot, dst).wait_send()
            acc_ref[slot] = blk
            send_desc(slot, dst).start()

        out_ref[pl.ds(my * m_per, m_per)] = compute_block(my)

        for s in range(max(1, N_DEV - NSLOTS), N_DEV):
            dst = lax.rem(my + s, N_DEV)
            send_desc((s - 1) % NSLOTS, dst).wait_send()

        for s in range(1, N_DEV):
            src = lax.rem(my + s, N_DEV)
            recv = pltpu.make_async_remote_copy(
                src_ref=acc_ref.at[0],
                dst_ref=out_ref.at[pl.ds(src * m_per, m_per)],
                send_sem=send_sems.at[0],
                recv_sem=recv_sems.at[src],
                device_id=(src,),
                device_id_type=pl.DeviceIdType.MESH,
            )
            recv.wait_recv()

    return pl.pallas_call(
        body,
        out_shape=jax.ShapeDtypeStruct((m, n_per), jnp.float32),
        in_specs=[
            pl.BlockSpec(memory_space=pltpu.VMEM),
            pl.BlockSpec(memory_space=pltpu.VMEM),
            pl.BlockSpec(memory_space=pltpu.SMEM),
            pl.BlockSpec(memory_space=pltpu.SMEM),
        ],
        out_specs=pl.BlockSpec(memory_space=pltpu.VMEM),
        scratch_shapes=[
            pltpu.VMEM((NSLOTS, m_per, n_per), jnp.float32),
            pltpu.SemaphoreType.DMA((NSLOTS,)),
            pltpu.SemaphoreType.DMA((N_DEV,)),
        ],
        compiler_params=pltpu.CompilerParams(
            collective_id=0,
            vmem_limit_bytes=100 * 1024 * 1024,
        ),
    )(x, w_mat, scale_x, scale_w)


# device time: 59566 ns/iter; 1.6568x vs baseline; 1.6568x over previous
import jax
import jax.numpy as jnp
from jax import lax
from jax.experimental import pallas as pl
from jax.experimental.pallas import tpu as pltpu

N_DEV = 32
NSLOTS = 8


def kernel(x, w_mat, scale_x, scale_w):
    m_per, k = x.shape
    _, n = w_mat.shape
    n_per = n // N_DEV
    m = m_per * N_DEV

    def body(x_ref, w_ref, sx_ref, sw_ref, out_ref,
             acc_ref, recv_ref, send_sems, recv_sems):
        my = lax.axis_index("i")
        scale = sx_ref[0] * sw_ref[0]

        barrier = pltpu.get_barrier_semaphore()
        for s in range(1, N_DEV):
            peer = lax.rem(my + s, N_DEV)
            pl.semaphore_signal(
                barrier, inc=1,
                device_id=(peer,), device_id_type=pl.DeviceIdType.MESH,
            )
        pl.semaphore_wait(barrier, N_DEV - 1)

        def compute_block(dst):
            wblk = w_ref[:, pl.ds(dst * n_per, n_per)]
            acc = lax.dot_general(
                x_ref[:, :], wblk,
                dimension_numbers=(((1,), (0,)), ((), ())),
                preferred_element_type=jnp.int32,
            )
            return (acc.astype(jnp.float32) * scale).astype(jnp.bfloat16)

        def send_desc(slot, dst):
            return pltpu.make_async_remote_copy(
                src_ref=acc_ref.at[slot],
                dst_ref=recv_ref.at[pl.ds(my * m_per, m_per)],
                send_sem=send_sems.at[slot],
                recv_sem=recv_sems.at[my],
                device_id=(dst,),
                device_id_type=pl.DeviceIdType.MESH,
            )

        for s in range(1, N_DEV):
            dst = lax.rem(my + s, N_DEV)
            blk = compute_block(dst)
            slot = (s - 1) % NSLOTS
            if s - 1 >= NSLOTS:
                send_desc(slot, dst).wait_send()
            acc_ref[slot] = blk
            send_desc(slot, dst).start()

        recv_ref[pl.ds(my * m_per, m_per)] = compute_block(my)

        for s in range(max(1, N_DEV - NSLOTS), N_DEV):
            dst = lax.rem(my + s, N_DEV)
            send_desc((s - 1) % NSLOTS, dst).wait_send()

        for s in range(1, N_DEV):
            src = lax.rem(my + s, N_DEV)
            recv = pltpu.make_async_remote_copy(
                src_ref=acc_ref.at[0],
                dst_ref=recv_ref.at[pl.ds(src * m_per, m_per)],
                send_sem=send_sems.at[0],
                recv_sem=recv_sems.at[src],
                device_id=(src,),
                device_id_type=pl.DeviceIdType.MESH,
            )
            recv.wait_recv()

        out_ref[:, :] = recv_ref[:, :].astype(jnp.float32)

    return pl.pallas_call(
        body,
        out_shape=jax.ShapeDtypeStruct((m, n_per), jnp.float32),
        in_specs=[
            pl.BlockSpec(memory_space=pltpu.VMEM),
            pl.BlockSpec(memory_space=pltpu.VMEM),
            pl.BlockSpec(memory_space=pltpu.SMEM),
            pl.BlockSpec(memory_space=pltpu.SMEM),
        ],
        out_specs=pl.BlockSpec(memory_space=pltpu.VMEM),
        scratch_shapes=[
            pltpu.VMEM((NSLOTS, m_per, n_per), jnp.bfloat16),
            pltpu.VMEM((m, n_per), jnp.bfloat16),
            pltpu.SemaphoreType.DMA((NSLOTS,)),
            pltpu.SemaphoreType.DMA((N_DEV,)),
        ],
        compiler_params=pltpu.CompilerParams(
            collective_id=0,
            vmem_limit_bytes=100 * 1024 * 1024,
        ),
    )(x, w_mat, scale_x, scale_w)


# device time: 23749 ns/iter; 4.1555x vs baseline; 2.5081x over previous
import jax
import jax.numpy as jnp
from jax import lax
from jax.experimental import pallas as pl
from jax.experimental.pallas import tpu as pltpu

N_DEV = 32
NSLOTS = 8

import os
_NO_COMM = os.environ.get("KERNEL_NO_COMM") == "1"


def kernel(x, w_mat, scale_x, scale_w):
    m_per, k = x.shape
    _, n = w_mat.shape
    n_per = n // N_DEV
    m = m_per * N_DEV

    def body(x_ref, w_ref, sx_ref, sw_ref, out_ref,
             acc_ref, recv_ref, send_sems, recv_sems):
        my = lax.axis_index("i")
        scale = sx_ref[0] * sw_ref[0]

        if not _NO_COMM:
            with jax.named_scope("entry_barrier"):
                barrier = pltpu.get_barrier_semaphore()
                for s in range(1, N_DEV):
                    peer = lax.rem(my + s, N_DEV)
                    pl.semaphore_signal(
                        barrier, inc=1,
                        device_id=(peer,), device_id_type=pl.DeviceIdType.MESH,
                    )
                pl.semaphore_wait(barrier, N_DEV - 1)

        def compute_block(dst):
            wblk = w_ref[:, pl.ds(dst * n_per, n_per)]
            acc = lax.dot_general(
                x_ref[:, :], wblk,
                dimension_numbers=(((1,), (0,)), ((), ())),
                preferred_element_type=jnp.int32,
            )
            return (acc.astype(jnp.float32) * scale).astype(jnp.bfloat16)

        def send_desc(slot, dst):
            return pltpu.make_async_remote_copy(
                src_ref=acc_ref.at[slot],
                dst_ref=recv_ref.at[pl.ds(my * m_per, m_per)],
                send_sem=send_sems.at[slot],
                recv_sem=recv_sems.at[my],
                device_id=(dst,),
                device_id_type=pl.DeviceIdType.MESH,
            )

        with jax.named_scope("gemm_send_loop"):
            for s in range(1, N_DEV):
                dst = lax.rem(my + s, N_DEV)
                blk = compute_block(dst)
                slot = (s - 1) % NSLOTS
                if not _NO_COMM and s - 1 >= NSLOTS:
                    send_desc(slot, dst).wait_send()
                acc_ref[slot] = blk
                if not _NO_COMM:
                    send_desc(slot, dst).start()

        with jax.named_scope("diag_block"):
            recv_ref[pl.ds(my * m_per, m_per)] = compute_block(my)

        if not _NO_COMM:
            with jax.named_scope("drain_send"):
                for s in range(max(1, N_DEV - NSLOTS), N_DEV):
                    dst = lax.rem(my + s, N_DEV)
                    send_desc((s - 1) % NSLOTS, dst).wait_send()

            with jax.named_scope("wait_recv"):
                for s in range(1, N_DEV):
                    src = lax.rem(my + s, N_DEV)
                    recv = pltpu.make_async_remote_copy(
                        src_ref=acc_ref.at[0],
                        dst_ref=recv_ref.at[pl.ds(src * m_per, m_per)],
                        send_sem=send_sems.at[0],
                        recv_sem=recv_sems.at[src],
                        device_id=(src,),
                        device_id_type=pl.DeviceIdType.MESH,
                    )
                    recv.wait_recv()

        with jax.named_scope("convert_f32"):
            out_ref[:, :] = recv_ref[:, :].astype(jnp.float32)

    return pl.pallas_call(
        body,
        out_shape=jax.ShapeDtypeStruct((m, n_per), jnp.float32),
        in_specs=[
            pl.BlockSpec(memory_space=pltpu.VMEM),
            pl.BlockSpec(memory_space=pltpu.VMEM),
            pl.BlockSpec(memory_space=pltpu.SMEM),
            pl.BlockSpec(memory_space=pltpu.SMEM),
        ],
        out_specs=pl.BlockSpec(memory_space=pltpu.VMEM),
        scratch_shapes=[
            pltpu.VMEM((NSLOTS, m_per, n_per), jnp.bfloat16),
            pltpu.VMEM((m, n_per), jnp.bfloat16),
            pltpu.SemaphoreType.DMA((NSLOTS,)),
            pltpu.SemaphoreType.DMA((N_DEV,)),
        ],
        compiler_params=pltpu.CompilerParams(
            collective_id=None if _NO_COMM else 0,
            vmem_limit_bytes=100 * 1024 * 1024,
        ),
    )(x, w_mat, scale_x, scale_w)
